# Initial kernel scaffold; baseline (speedup 1.0000x reference)
import jax
import jax.numpy as jnp
from jax.experimental import pallas as pl


def kernel(x, W1, b1, g1, be1, Wg, bg, gg, beg, W2, b2, g2, be2):
    raise NotImplementedError("write your pallas kernel here")


# speedup vs baseline: 5.1596x; 5.1596x over previous
import functools

import jax
import jax.numpy as jnp
from jax.experimental import pallas as pl

_EPS = 1e-5
_KNN = 9
_HI = jax.lax.Precision.HIGHEST
_INV_SQRT2 = 0.7071067811865476


def _dot(a, b, dims, precision=_HI):
    return jax.lax.dot_general(a, b, (dims, ((), ())),
                               preferred_element_type=jnp.float32,
                               precision=precision)


def _accum_stats(s_ref, val, is_first):
    st = jnp.concatenate([jnp.sum(val, axis=0, keepdims=True),
                          jnp.sum(val * val, axis=0, keepdims=True)], axis=0)

    @pl.when(is_first)
    def _():
        s_ref[...] = jnp.zeros_like(s_ref)

    s_ref[...] += st


def _affine_from_stats(s_ref, g_ref, be_ref, n_tot):
    inv = 1.0 / n_tot
    mean = s_ref[0:1, :] * inv
    var = s_ref[1:2, :] * inv - mean * mean
    a = g_ref[...] * jax.lax.rsqrt(var + _EPS)
    d = be_ref[...] - mean * a
    return a, d


def _conv1_body(xt_ref, w1_ref, b1_ref, h_ref, s_ref):
    b = pl.program_id(0)
    h = _dot(xt_ref[0], w1_ref[...], ((1,), (1,)), precision=None) + b1_ref[...]
    h_ref[0] = h
    _accum_stats(s_ref, h, b == 0)


def _graph_body(n_tot, h_ref, s1_ref, g1_ref, be1_ref, wxr_ref, wdf_ref,
                bg_ref, zg_ref, s2_ref):
    b = pl.program_id(0)
    a, d = _affine_from_stats(s1_ref, g1_ref, be1_ref, n_tot)
    xr = h_ref[0] * a + d
    nsq = jnp.sum(xr * xr, axis=1, keepdims=True)
    xn = xr * (1.0 / jnp.maximum(jnp.sqrt(nsq), 1e-12))
    n = xn.shape[0]
    sim = _dot(xn, xn, ((1,), (1,)), precision=None)
    sq_row = _dot(jnp.ones((8, xn.shape[1]), jnp.float32), xn * xn,
                  ((1,), (1,)))[0:1]
    v = sq_row - 2.0 * sim
    col = jax.lax.broadcasted_iota(jnp.int32, v.shape, 1)
    acc = None
    for k in range(_KNN):
        rowmin = jnp.min(v, axis=1, keepdims=True)
        idx = jnp.min(jnp.where(v == rowmin, col, n), axis=1, keepdims=True)
        e = col == idx
        g = _dot(e.astype(jnp.float32), xr, ((1,), (0,)))
        acc = g if acc is None else jnp.maximum(acc, g)
        v = jnp.where(e, jnp.inf, v)
    diff = acc - xr
    zg = (_dot(xr, wxr_ref[...], ((1,), (0,)))
          + _dot(diff, wdf_ref[...], ((1,), (0,))) + bg_ref[...])
    zg_ref[0] = zg
    _accum_stats(s2_ref, zg, b == 0)


def _head_body(n_tot, zg_ref, s2_ref, gg_ref, beg_ref, w2_ref, b2_ref,
               o_ref, s3_ref):
    b = pl.program_id(0)
    a, d = _affine_from_stats(s2_ref, gg_ref, beg_ref, n_tot)
    t = zg_ref[0] * a + d
    z = 0.5 * t * (1.0 + jax.lax.erf(t * _INV_SQRT2))
    o = _dot(z, w2_ref[...], ((1,), (1,))) + b2_ref[...]
    o_ref[0] = o
    _accum_stats(s3_ref, o, b == 0)


def _res_body(n_tot, o_ref, s3_ref, g2_ref, be2_ref, xt_ref, out_ref):
    a, d = _affine_from_stats(s3_ref, g2_ref, be2_ref, n_tot)
    out_ref[0] = o_ref[0] * a + d + xt_ref[0]


def _block_diag(wb, c_in, c_out, groups, dtype):
    ig, og = c_in // groups, c_out // groups
    m = jnp.zeros((c_in, c_out), dtype)
    for g in range(groups):
        m = m.at[g * ig:(g + 1) * ig, g * og:(g + 1) * og].set(
            jnp.transpose(wb[g]))
    return m


def kernel(x, W1, b1, g1, be1, Wg, bg, gg, beg, W2, b2, g2, be2):
    B, C, H, W = x.shape
    N = H * W
    Cout = Wg.shape[0]
    groups = 4
    n_tot = float(B * N)
    f32 = jnp.float32

    xt = jnp.transpose(x.reshape(B, C, N), (0, 2, 1))
    wg3 = Wg.reshape(groups, Cout // groups, (2 * C) // groups)
    wxr = _block_diag(wg3[:, :, 0::2], C, Cout, groups, f32)
    wdf = _block_diag(wg3[:, :, 1::2], C, Cout, groups, f32)

    row = lambda v: v.reshape(1, -1).astype(f32)
    b1r, g1r, be1r = row(b1), row(g1), row(be1)
    bgr, ggr, begr = row(bg), row(gg), row(beg)
    b2r, g2r, be2r = row(b2), row(g2), row(be2)

    full = lambda shape: pl.BlockSpec(shape, lambda b: (0,) * len(shape))
    per_b = lambda shape: pl.BlockSpec((1,) + shape,
                                       lambda b: (b,) + (0,) * len(shape))

    h_raw, s1 = pl.pallas_call(
        _conv1_body,
        grid=(B,),
        in_specs=[per_b((N, C)), full((C, C)), full((1, C))],
        out_specs=[per_b((N, C)), full((2, C))],
        out_shape=[jax.ShapeDtypeStruct((B, N, C), f32),
                   jax.ShapeDtypeStruct((2, C), f32)],
    )(xt, W1, b1r)

    zg_raw, s2 = pl.pallas_call(
        functools.partial(_graph_body, n_tot),
        grid=(B,),
        in_specs=[per_b((N, C)), full((2, C)), full((1, C)), full((1, C)),
                  full((C, Cout)), full((C, Cout)), full((1, Cout))],
        out_specs=[per_b((N, Cout)), full((2, Cout))],
        out_shape=[jax.ShapeDtypeStruct((B, N, Cout), f32),
                   jax.ShapeDtypeStruct((2, Cout), f32)],
    )(h_raw, s1, g1r, be1r, wxr, wdf, bgr)

    o_raw, s3 = pl.pallas_call(
        functools.partial(_head_body, n_tot),
        grid=(B,),
        in_specs=[per_b((N, Cout)), full((2, Cout)), full((1, Cout)),
                  full((1, Cout)), full((C, Cout)), full((1, C))],
        out_specs=[per_b((N, C)), full((2, C))],
        out_shape=[jax.ShapeDtypeStruct((B, N, C), f32),
                   jax.ShapeDtypeStruct((2, C), f32)],
    )(zg_raw, s2, ggr, begr, W2, b2r)

    out = pl.pallas_call(
        functools.partial(_res_body, n_tot),
        grid=(B,),
        in_specs=[per_b((N, C)), full((2, C)), full((1, C)), full((1, C)),
                  per_b((N, C))],
        out_specs=per_b((N, C)),
        out_shape=jax.ShapeDtypeStruct((B, N, C), f32),
    )(o_raw, s3, g2r, be2r, xt)

    return jnp.transpose(out, (0, 2, 1)).reshape(B, C, H, W)


# speedup vs baseline: 11.4842x; 2.2258x over previous
import functools

import jax
import jax.numpy as jnp
from jax.experimental import pallas as pl

_EPS = 1e-5
_KNN = 9
_HI = jax.lax.Precision.HIGHEST
_INV_SQRT2 = 0.7071067811865476


def _dot(a, b, dims, precision=_HI):
    return jax.lax.dot_general(a, b, (dims, ((), ())),
                               preferred_element_type=jnp.float32,
                               precision=precision)


def _accum_stats(s_ref, val, is_first):
    st = jnp.concatenate([jnp.sum(val, axis=0, keepdims=True),
                          jnp.sum(val * val, axis=0, keepdims=True)], axis=0)

    @pl.when(is_first)
    def _():
        s_ref[...] = jnp.zeros_like(s_ref)

    s_ref[...] += st


def _affine_from_stats(s_ref, g_ref, be_ref, n_tot):
    inv = 1.0 / n_tot
    mean = s_ref[0:1, :] * inv
    var = s_ref[1:2, :] * inv - mean * mean
    a = g_ref[...] * jax.lax.rsqrt(var + _EPS)
    d = be_ref[...] - mean * a
    return a, d


def _conv1_body(xt_ref, w1_ref, b1_ref, h_ref, s_ref):
    b = pl.program_id(0)
    h = _dot(xt_ref[0], w1_ref[...], ((1,), (1,)), precision=None) + b1_ref[...]
    h_ref[0] = h
    _accum_stats(s_ref, h, b == 0)


def _graph_body(n_tot, h_ref, s1_ref, g1_ref, be1_ref, wxr_ref, wdf_ref,
                bg_ref, zg_ref, s2_ref):
    b = pl.program_id(0)
    a, d = _affine_from_stats(s1_ref, g1_ref, be1_ref, n_tot)
    xr = h_ref[0] * a + d
    nsq = jnp.sum(xr * xr, axis=1, keepdims=True)
    xn = xr * (1.0 / jnp.maximum(jnp.sqrt(nsq), 1e-12))
    n = xn.shape[0]
    sim = _dot(xn, xn, ((1,), (1,)), precision=None)
    sq_row = _dot(jnp.ones((8, xn.shape[1]), jnp.float32), xn * xn,
                  ((1,), (1,)))[0:1]
    v = sq_row - 2.0 * sim
    col = jax.lax.broadcasted_iota(jnp.int32, v.shape, 1)
    xr_hi = xr.astype(jnp.bfloat16)
    xr_lo = (xr - xr_hi.astype(jnp.float32)).astype(jnp.bfloat16)
    acc = None
    for k in range(_KNN):
        rowmin = jnp.min(v, axis=1, keepdims=True)
        idx = jnp.min(jnp.where(v == rowmin, col, n), axis=1, keepdims=True)
        e = col == idx
        eb = e.astype(jnp.bfloat16)
        g = (_dot(eb, xr_hi, ((1,), (0,)), precision=None)
             + _dot(eb, xr_lo, ((1,), (0,)), precision=None))
        acc = g if acc is None else jnp.maximum(acc, g)
        v = jnp.where(e, jnp.inf, v)
    diff = acc - xr
    zg = (_dot(xr, wxr_ref[...], ((1,), (0,)))
          + _dot(diff, wdf_ref[...], ((1,), (0,))) + bg_ref[...])
    zg_ref[0] = zg
    _accum_stats(s2_ref, zg, b == 0)


def _head_body(n_tot, zg_ref, s2_ref, gg_ref, beg_ref, w2_ref, b2_ref,
               o_ref, s3_ref):
    b = pl.program_id(0)
    a, d = _affine_from_stats(s2_ref, gg_ref, beg_ref, n_tot)
    t = zg_ref[0] * a + d
    z = 0.5 * t * (1.0 + jax.lax.erf(t * _INV_SQRT2))
    o = _dot(z, w2_ref[...], ((1,), (1,))) + b2_ref[...]
    o_ref[0] = o
    _accum_stats(s3_ref, o, b == 0)


def _res_body(n_tot, o_ref, s3_ref, g2_ref, be2_ref, xt_ref, out_ref):
    a, d = _affine_from_stats(s3_ref, g2_ref, be2_ref, n_tot)
    out_ref[0] = o_ref[0] * a + d + xt_ref[0]


def _block_diag(wb, c_in, c_out, groups, dtype):
    ig, og = c_in // groups, c_out // groups
    m = jnp.zeros((c_in, c_out), dtype)
    for g in range(groups):
        m = m.at[g * ig:(g + 1) * ig, g * og:(g + 1) * og].set(
            jnp.transpose(wb[g]))
    return m


def kernel(x, W1, b1, g1, be1, Wg, bg, gg, beg, W2, b2, g2, be2):
    B, C, H, W = x.shape
    N = H * W
    Cout = Wg.shape[0]
    groups = 4
    n_tot = float(B * N)
    f32 = jnp.float32

    xt = jnp.transpose(x.reshape(B, C, N), (0, 2, 1))
    wg3 = Wg.reshape(groups, Cout // groups, (2 * C) // groups)
    wxr = _block_diag(wg3[:, :, 0::2], C, Cout, groups, f32)
    wdf = _block_diag(wg3[:, :, 1::2], C, Cout, groups, f32)

    row = lambda v: v.reshape(1, -1).astype(f32)
    b1r, g1r, be1r = row(b1), row(g1), row(be1)
    bgr, ggr, begr = row(bg), row(gg), row(beg)
    b2r, g2r, be2r = row(b2), row(g2), row(be2)

    full = lambda shape: pl.BlockSpec(shape, lambda b: (0,) * len(shape))
    per_b = lambda shape: pl.BlockSpec((1,) + shape,
                                       lambda b: (b,) + (0,) * len(shape))

    h_raw, s1 = pl.pallas_call(
        _conv1_body,
        grid=(B,),
        in_specs=[per_b((N, C)), full((C, C)), full((1, C))],
        out_specs=[per_b((N, C)), full((2, C))],
        out_shape=[jax.ShapeDtypeStruct((B, N, C), f32),
                   jax.ShapeDtypeStruct((2, C), f32)],
    )(xt, W1, b1r)

    zg_raw, s2 = pl.pallas_call(
        functools.partial(_graph_body, n_tot),
        grid=(B,),
        in_specs=[per_b((N, C)), full((2, C)), full((1, C)), full((1, C)),
                  full((C, Cout)), full((C, Cout)), full((1, Cout))],
        out_specs=[per_b((N, Cout)), full((2, Cout))],
        out_shape=[jax.ShapeDtypeStruct((B, N, Cout), f32),
                   jax.ShapeDtypeStruct((2, Cout), f32)],
    )(h_raw, s1, g1r, be1r, wxr, wdf, bgr)

    o_raw, s3 = pl.pallas_call(
        functools.partial(_head_body, n_tot),
        grid=(B,),
        in_specs=[per_b((N, Cout)), full((2, Cout)), full((1, Cout)),
                  full((1, Cout)), full((C, Cout)), full((1, C))],
        out_specs=[per_b((N, C)), full((2, C))],
        out_shape=[jax.ShapeDtypeStruct((B, N, C), f32),
                   jax.ShapeDtypeStruct((2, C), f32)],
    )(zg_raw, s2, ggr, begr, W2, b2r)

    out = pl.pallas_call(
        functools.partial(_res_body, n_tot),
        grid=(B,),
        in_specs=[per_b((N, C)), full((2, C)), full((1, C)), full((1, C)),
                  per_b((N, C))],
        out_specs=per_b((N, C)),
        out_shape=jax.ShapeDtypeStruct((B, N, C), f32),
    )(o_raw, s3, g2r, be2r, xt)

    return jnp.transpose(out, (0, 2, 1)).reshape(B, C, H, W)
